# baseline (speedup 1.0000x reference)
import jax
import jax.lax as lax
import jax.numpy as jnp
import numpy as np
from jax.experimental import pallas as pl

_SLOTS = 1024
_DIM = 64
_GROUPS = 2
_TEMP = 0.4
_LOG_SLOTS = float(np.log(_SLOTS))
_TINY = float(np.finfo(np.float32).tiny)

_ROT_A = (13, 15, 26, 6)
_ROT_B = (17, 29, 16, 24)
_KS0 = np.uint32(0)
_KS1 = np.uint32(42)
_KS2 = np.uint32(0x1BD11BDA ^ 42)


def _tf_rounds(x0, x1, rots):
    for r in rots:
        x0 = x0 + x1
        x1 = (x1 << np.uint32(r)) | lax.shift_right_logical(
            x1, np.uint32(32 - r)
        )
        x1 = x1 ^ x0
    return x0, x1


def _gumbel_block(slab, w):
    k_iota = lax.broadcasted_iota(jnp.uint32, (_SLOTS, w), 0)
    t_iota = lax.broadcasted_iota(jnp.uint32, (_SLOTS, w), 1)
    base = (slab * (_SLOTS * w)).astype(jnp.uint32)
    lin = k_iota + t_iota * np.uint32(_SLOTS) + base
    x0 = jnp.zeros((_SLOTS, w), jnp.uint32)
    x1 = lin + _KS1
    x0, x1 = _tf_rounds(x0, x1, _ROT_A)
    x0 = x0 + _KS1
    x1 = x1 + (_KS2 + np.uint32(1))
    x0, x1 = _tf_rounds(x0, x1, _ROT_B)
    x0 = x0 + _KS2
    x1 = x1 + (_KS0 + np.uint32(2))
    x0, x1 = _tf_rounds(x0, x1, _ROT_A)
    x0 = x0 + _KS0
    x1 = x1 + (_KS1 + np.uint32(3))
    x0, x1 = _tf_rounds(x0, x1, _ROT_B)
    x0 = x0 + _KS1
    x1 = x1 + (_KS2 + np.uint32(4))
    x0, x1 = _tf_rounds(x0, x1, _ROT_A)
    x0 = x0 + _KS2
    x1 = x1 + (_KS0 + np.uint32(5))
    bits = x0 ^ x1
    fb = lax.shift_right_logical(bits, np.uint32(9)) | np.uint32(0x3F800000)
    f = lax.bitcast_convert_type(fb, jnp.float32) - 1.0
    u = jnp.maximum(f, jnp.float32(_TINY))
    return -jnp.log(-jnp.log(u))


def _vq_block(z_ref, cb_ref, zq_ref, idx_ref, s_ref, m_ref):
    z = z_ref[0]
    cb = cb_ref[...]
    w = z.shape[1]
    g = _gumbel_block(pl.program_id(0), w)

    mm = jax.lax.dot_general(
        cb, z, (((1,), (0,)), ((), ())), preferred_element_type=jnp.float32
    )
    cb_sqr = jnp.sum(cb * cb, axis=1)[:, None]
    z_sqr = jnp.sum(z * z, axis=0)[None, :]
    logits = 2.0 * mm - cb_sqr - z_sqr

    y = (logits + g) * (1.0 / _TEMP)
    y_max = jnp.max(y, axis=0, keepdims=True)
    e = jnp.exp(y - y_max)
    s = jnp.sum(e, axis=0, keepdims=True)
    idx_ref[0, 0] = jnp.argmax(y, axis=0)

    zq_un = jax.lax.dot_general(
        cb, e, (((0,), (0,)), ((), ())), preferred_element_type=jnp.float32
    )
    zq_ref[0] = zq_un / s

    m2 = jnp.max(logits, axis=0, keepdims=True)
    e2 = jnp.exp(logits - m2)
    s2 = jnp.sum(e2, axis=0, keepdims=True)
    t = jnp.sum(e2 * logits, axis=0, keepdims=True)
    s_part = jnp.sum(t / s2, axis=1, keepdims=True)
    m_part = jnp.sum(m2 + jnp.log(s2), axis=1, keepdims=True)

    @pl.when(pl.program_id(0) == 0)
    def _init():
        s_ref[...] = jnp.zeros((1, 1), jnp.float32)
        m_ref[...] = jnp.zeros((1, 1), jnp.float32)

    s_ref[...] += s_part
    m_ref[...] += m_part


def kernel(z_e, codebook):
    bs, feat_dim, w = z_e.shape
    n_slabs = bs * _GROUPS
    zr = z_e.reshape(n_slabs, _DIM, w)

    zq, idx, s_tot, m_tot = pl.pallas_call(
        _vq_block,
        grid=(n_slabs,),
        in_specs=[
            pl.BlockSpec((1, _DIM, w), lambda i: (i, 0, 0)),
            pl.BlockSpec((_SLOTS, _DIM), lambda i: (0, 0)),
        ],
        out_specs=[
            pl.BlockSpec((1, _DIM, w), lambda i: (i, 0, 0)),
            pl.BlockSpec((1, 1, w), lambda i: (i, 0, 0)),
            pl.BlockSpec((1, 1), lambda i: (0, 0)),
            pl.BlockSpec((1, 1), lambda i: (0, 0)),
        ],
        out_shape=[
            jax.ShapeDtypeStruct((n_slabs, _DIM, w), jnp.float32),
            jax.ShapeDtypeStruct((n_slabs, 1, w), jnp.int32),
            jax.ShapeDtypeStruct((1, 1), jnp.float32),
            jax.ShapeDtypeStruct((1, 1), jnp.float32),
        ],
    )(zr, codebook)

    n_rows = n_slabs * w
    denom = float(n_rows * _SLOTS)
    s0 = s_tot[0, 0]
    kl = (s0 - m_tot[0, 0] + n_rows * _LOG_SLOTS) / denom
    commit = -s0 / denom
    z_q = zq.reshape(bs, feat_dim, w)
    hard_indices = idx.reshape(bs, _GROUPS, w)
    return (z_q, hard_indices, kl, commit)
